# baseline (speedup 1.0000x reference)
import functools

import jax
import jax.numpy as jnp
from jax import lax
from jax.experimental import pallas as pl
from jax.experimental.pallas import tpu as pltpu
from jax.experimental.pallas import tpu_sc as plsc

K = 8192
D = 32
N = 8192
TB = 256
GRID = N // TB
LOG2K = 13.0
TEMP = 1.0
LOG2E = 1.4426950408889634
LN2 = 0.6931471805599453

NW = 16
BW = N // NW
TABN = 8200


def _vq_tc_kernel(x_ref, emb_ref, cc_ref, idx_ref, scal_ref, aug_ref,
                  m2e_ref):
    i = pl.program_id(0)

    @pl.when(i == 0)
    def _init():
        emb = emb_ref[...]
        e2 = jnp.sum(emb * emb, axis=1, keepdims=True)
        aug_ref[...] = jnp.broadcast_to(e2, (K, TB))
        m2e_ref[...] = -2.0 * emb
        scal_ref[2] = 0.0
        cc = cc_ref[...]
        maskf = (cc >= 1.0).astype(jnp.float32)
        n_used = jnp.sum(maskf)
        norms = jnp.maximum(jnp.sqrt(e2), 1e-12)
        nemb = (emb / norms) * maskf
        gram = lax.dot_general(nemb, nemb, (((0,), (0,)), ((), ())),
                               preferred_element_type=jnp.float32)
        scal_ref[1] = (jnp.sum(gram * gram) / (n_used * n_used)
                       - 1.0 / n_used)
        scal_ref[4] = n_used / K

    x = x_ref[...]
    x2 = lax.dot_general(jnp.ones((1, D), jnp.float32), x * x,
                         (((1,), (1,)), ((), ())),
                         preferred_element_type=jnp.float32)
    xe2 = lax.dot_general(m2e_ref[...], x, (((1,), (1,)), ((), ())),
                          preferred_element_type=jnp.float32)
    d = (x2 + aug_ref[...]) + xe2
    idx = jnp.argmin(d, axis=0, keepdims=True).astype(jnp.int32)
    idx_ref[...] = idx[None]

    c = jnp.minimum(x2, 60.0)
    cd = c - d
    el = jnp.exp(cd)
    ones_k = jnp.ones((1, K), jnp.bfloat16)
    s = lax.dot_general(ones_k, el.astype(jnp.bfloat16),
                        (((1,), (0,)), ((), ())),
                        preferred_element_type=jnp.float32)
    t2 = lax.dot_general(ones_k, (el * cd).astype(jnp.bfloat16),
                         (((1,), (0,)), ((), ())),
                         preferred_element_type=jnp.float32)
    ent = jnp.log(s) - t2 / s
    scal_ref[2] += jnp.sum(ent)

    @pl.when(i == GRID - 1)
    def _finalize():
        scal_ref[2] = scal_ref[2] / (N * LOG2K)


def _vq_tc(flat_x, embedding, cc_col, interpret=False):
    idx3, scal = pl.pallas_call(
        _vq_tc_kernel,
        grid=(GRID,),
        in_specs=[
            pl.BlockSpec((TB, D), lambda i: (i, 0)),
            pl.BlockSpec((K, D), lambda i: (0, 0)),
            pl.BlockSpec((K, 1), lambda i: (0, 0)),
        ],
        out_specs=[
            pl.BlockSpec((1, 1, TB), lambda i: (i, 0, 0)),
            pl.BlockSpec(memory_space=pltpu.SMEM),
        ],
        out_shape=[
            jax.ShapeDtypeStruct((GRID, 1, TB), jnp.int32),
            jax.ShapeDtypeStruct((8,), jnp.float32),
        ],
        scratch_shapes=[pltpu.VMEM((K, TB), jnp.float32),
                        pltpu.VMEM((K, D), jnp.float32)],
        interpret=interpret,
    )(flat_x, embedding, cc_col)
    return idx3.reshape(N), scal


def _make_sc_kernel():
    mesh = plsc.VectorSubcoreMesh(core_axis_name="c", subcore_axis_name="s",
                                  num_cores=1)

    @functools.partial(
        pl.kernel,
        out_type=[
            jax.ShapeDtypeStruct((N, D), jnp.float32),
            jax.ShapeDtypeStruct((16,), jnp.float32),
            jax.ShapeDtypeStruct((16,), jnp.float32),
        ],
        mesh=mesh,
        scratch_types=[
            pltpu.VMEM((BW,), jnp.int32),
            pltpu.VMEM((BW, D), jnp.float32),
            pltpu.VMEM((BW, D), jnp.float32),
            pltpu.VMEM((K // 128, 128), jnp.float32),
            pltpu.VMEM((TABN,), jnp.float32),
            pltpu.VMEM((BW // 128, 128), jnp.float32),
            pltpu.VMEM((16,), jnp.float32),
            pltpu.VMEM_SHARED((K // 128, 128), jnp.float32),
            pltpu.VMEM_SHARED((NW, 16), jnp.float32),
            pltpu.VMEM_SHARED((NW, 16), jnp.float32),
            pltpu.SemaphoreType.DMA,
        ],
        compiler_params=pltpu.CompilerParams(use_tc_tiling_on_sc=False,
                                             needs_layout_passes=False),
    )
    def sc_kernel(table_hbm, idx_hbm, x_hbm, tab_hbm, zz_hbm,
                  out_hbm, commit_hbm, perp_hbm,
                  idx_v, rows_v, x_v, hist_v, tab_v, cnt_v, stage_v,
                  counts_sh, cpart_sh, ppart_sh, sem):
        wid = lax.axis_index("s")
        base = wid * BW

        pltpu.sync_copy(idx_hbm.at[pl.ds(base, BW)], idx_v)
        pltpu.sync_copy(zz_hbm, hist_v)
        pltpu.async_copy(table_hbm.at[idx_v], rows_v, sem).wait()
        pltpu.sync_copy(rows_v, out_hbm.at[pl.ds(base, BW)])
        pltpu.sync_copy(x_hbm.at[pl.ds(base, BW)], x_v)
        pltpu.sync_copy(tab_hbm, tab_v)

        def commit_vec(j, acc):
            a = rows_v[j, pl.ds(0, 16)] - x_v[j, pl.ds(0, 16)]
            b = rows_v[j, pl.ds(16, 16)] - x_v[j, pl.ds(16, 16)]
            return acc + (a * a + b * b)
        cacc = lax.fori_loop(0, BW, commit_vec,
                             jnp.zeros((16,), jnp.float32))

        ones16 = jnp.ones((16,), jnp.float32)

        def hist_body(j, _):
            ivec = idx_v[pl.ds(j * 16, 16)]
            plsc.addupdate_scatter(
                hist_v, [jnp.right_shift(ivec, 7), ivec & 127], ones16)
            return 0
        lax.fori_loop(0, BW // 16, hist_body, 0)

        @pl.when(wid == 0)
        def _seed():
            pltpu.sync_copy(hist_v, counts_sh)
        plsc.subcore_barrier()

        @pl.when(wid != 0)
        def _add():
            for jj in range(K // 128 // 16):
                rows = lax.iota(jnp.int32, 16) + jj * 16
                pltpu.sync_copy(hist_v.at[pl.ds(jj * 16, 16)],
                                counts_sh.at[rows], add=True)
        stage_v[...] = cacc
        pltpu.sync_copy(stage_v, cpart_sh.at[wid])
        plsc.subcore_barrier()

        pltpu.sync_copy(counts_sh.at[pl.ds(wid * (BW // 128), BW // 128)],
                        cnt_v)

        def perp_vec(j, acc):
            cv = cnt_v[j >> 3, pl.ds((j & 7) * 16, 16)]
            ci = cv.astype(jnp.int32)
            tv = plsc.load_gather(tab_v, [ci])
            return acc + cv * tv
        pacc = lax.fori_loop(0, BW // 16, perp_vec,
                             jnp.zeros((16,), jnp.float32))
        stage_v[...] = pacc
        pltpu.sync_copy(stage_v, ppart_sh.at[wid])
        plsc.subcore_barrier()

        @pl.when(wid == 0)
        def _final():
            def red(j, accs):
                ca, pa = accs
                pltpu.sync_copy(cpart_sh.at[j], stage_v)
                ca = ca + stage_v[...]
                pltpu.sync_copy(ppart_sh.at[j], stage_v)
                return (ca, pa + stage_v[...])
            ca, pa = lax.fori_loop(0, NW, red,
                                   (jnp.zeros((16,), jnp.float32),
                                    jnp.zeros((16,), jnp.float32)))
            commit = jnp.sum(ca) * (1.0 / (N * D))
            stage_v[...] = jnp.broadcast_to(commit, (16,))
            pltpu.sync_copy(stage_v, commit_hbm)
            parg = jnp.broadcast_to(-jnp.sum(pa) * (1.0 / N), (16,))
            stage_v[...] = jnp.exp(parg)
            pltpu.sync_copy(stage_v, perp_hbm)

    return sc_kernel


def kernel(inputs, embedding, code_count):
    input_shape = inputs.shape
    flat_x = inputs.reshape(N, D)
    cc_col = code_count.reshape(K, 1)
    idx, scal = _vq_tc(flat_x, embedding, cc_col)
    tab = jnp.log(jnp.arange(TABN, dtype=jnp.float32) * (1.0 / N) + 1e-10)
    zz = jnp.zeros((K // 128, 128), jnp.float32)
    quantized, commit_v, perp_v = _make_sc_kernel()(embedding, idx, flat_x,
                                                    tab, zz)
    quantized_st = quantized.reshape(input_shape)
    commitment_loss = commit_v[0]
    ortho_loss = scal[1]
    entropy_loss = scal[2]
    perplexity = perp_v[0]
    coverage = scal[4]
    return (commitment_loss, ortho_loss, entropy_loss, perplexity, coverage,
            quantized_st, idx.reshape(input_shape[0], -1))


# speedup vs baseline: 1.0215x; 1.0215x over previous
import functools

import jax
import jax.numpy as jnp
from jax import lax
from jax.experimental import pallas as pl
from jax.experimental.pallas import tpu as pltpu
from jax.experimental.pallas import tpu_sc as plsc

K = 8192
D = 32
N = 8192
TB = 256
GRID = N // TB
LOG2K = 13.0
TEMP = 1.0
LOG2E = 1.4426950408889634
LN2 = 0.6931471805599453

NW = 16
BW = N // NW
TABN = 8200


def _vq_tc_kernel(x_ref, emb_ref, cc_ref, idx_ref, scal_ref, aug_ref,
                  m2e_ref):
    i = pl.program_id(0)

    @pl.when(i == 0)
    def _init():
        emb = emb_ref[...]
        e2 = jnp.sum(emb * emb, axis=1, keepdims=True)
        aug_ref[...] = jnp.broadcast_to(e2, (K, TB))
        m2e_ref[...] = -2.0 * emb
        scal_ref[2] = 0.0
        cc = cc_ref[...]
        maskf = (cc >= 1.0).astype(jnp.float32)
        n_used = jnp.sum(maskf)
        norms = jnp.maximum(jnp.sqrt(e2), 1e-12)
        nemb = (emb / norms) * maskf
        gram = lax.dot_general(nemb, nemb, (((0,), (0,)), ((), ())),
                               preferred_element_type=jnp.float32)
        scal_ref[1] = (jnp.sum(gram * gram) / (n_used * n_used)
                       - 1.0 / n_used)
        scal_ref[4] = n_used / K

    x = x_ref[...]
    x2 = lax.dot_general(jnp.ones((1, D), jnp.float32), x * x,
                         (((1,), (1,)), ((), ())),
                         preferred_element_type=jnp.float32)
    xe2 = lax.dot_general(m2e_ref[...], x, (((1,), (1,)), ((), ())),
                          preferred_element_type=jnp.float32)
    d = (x2 + aug_ref[...]) + xe2
    idx = jnp.argmin(d, axis=0, keepdims=True).astype(jnp.int32)
    idx_ref[...] = idx[None]

    c = jnp.minimum(x2, 60.0)
    md2 = (c - d) * (LOG2E / TEMP)
    el = jnp.exp2(md2)
    ones_k = jnp.ones((1, K), jnp.float32)
    s = lax.dot_general(ones_k, el, (((1,), (0,)), ((), ())),
                        preferred_element_type=jnp.float32)
    t2 = lax.dot_general(ones_k, el * md2, (((1,), (0,)), ((), ())),
                         preferred_element_type=jnp.float32)
    ent = jnp.log(s) - LN2 * (t2 / s)
    scal_ref[2] += jnp.sum(ent)

    @pl.when(i == GRID - 1)
    def _finalize():
        scal_ref[2] = scal_ref[2] / (N * LOG2K)


def _vq_tc(flat_x, embedding, cc_col, interpret=False):
    idx3, scal = pl.pallas_call(
        _vq_tc_kernel,
        grid=(GRID,),
        in_specs=[
            pl.BlockSpec((TB, D), lambda i: (i, 0)),
            pl.BlockSpec((K, D), lambda i: (0, 0)),
            pl.BlockSpec((K, 1), lambda i: (0, 0)),
        ],
        out_specs=[
            pl.BlockSpec((1, 1, TB), lambda i: (i, 0, 0)),
            pl.BlockSpec(memory_space=pltpu.SMEM),
        ],
        out_shape=[
            jax.ShapeDtypeStruct((GRID, 1, TB), jnp.int32),
            jax.ShapeDtypeStruct((8,), jnp.float32),
        ],
        scratch_shapes=[pltpu.VMEM((K, TB), jnp.float32),
                        pltpu.VMEM((K, D), jnp.float32)],
        interpret=interpret,
    )(flat_x, embedding, cc_col)
    return idx3.reshape(N), scal


def _make_sc_kernel():
    mesh = plsc.VectorSubcoreMesh(core_axis_name="c", subcore_axis_name="s",
                                  num_cores=1)

    @functools.partial(
        pl.kernel,
        out_type=[
            jax.ShapeDtypeStruct((N, D), jnp.float32),
            jax.ShapeDtypeStruct((16,), jnp.float32),
            jax.ShapeDtypeStruct((16,), jnp.float32),
        ],
        mesh=mesh,
        scratch_types=[
            pltpu.VMEM((BW,), jnp.int32),
            pltpu.VMEM((BW, D), jnp.float32),
            pltpu.VMEM((BW, D), jnp.float32),
            pltpu.VMEM((K // 128, 128), jnp.float32),
            pltpu.VMEM((TABN,), jnp.float32),
            pltpu.VMEM((BW // 128, 128), jnp.float32),
            pltpu.VMEM((16,), jnp.float32),
            pltpu.VMEM((NW, 16), jnp.float32),
            pltpu.VMEM_SHARED((K // 128, 128), jnp.float32),
            pltpu.VMEM_SHARED((NW, 16), jnp.float32),
            pltpu.VMEM_SHARED((NW, 16), jnp.float32),
            pltpu.SemaphoreType.DMA,
        ],
        compiler_params=pltpu.CompilerParams(use_tc_tiling_on_sc=False,
                                             needs_layout_passes=False),
    )
    def sc_kernel(table_hbm, idx_hbm, x_hbm, tab_hbm, zz_hbm,
                  out_hbm, commit_hbm, perp_hbm,
                  idx_v, rows_v, x_v, hist_v, tab_v, cnt_v, stage_v, bulk_v,
                  counts_sh, cpart_sh, ppart_sh, sem):
        wid = lax.axis_index("s")
        base = wid * BW

        pltpu.sync_copy(idx_hbm.at[pl.ds(base, BW)], idx_v)
        pltpu.sync_copy(zz_hbm, hist_v)
        pltpu.async_copy(table_hbm.at[idx_v], rows_v, sem).wait()
        pltpu.sync_copy(rows_v, out_hbm.at[pl.ds(base, BW)])
        pltpu.sync_copy(x_hbm.at[pl.ds(base, BW)], x_v)
        pltpu.sync_copy(tab_hbm, tab_v)

        def commit_vec(j, acc):
            a = rows_v[j, pl.ds(0, 16)] - x_v[j, pl.ds(0, 16)]
            b = rows_v[j, pl.ds(16, 16)] - x_v[j, pl.ds(16, 16)]
            return acc + (a * a + b * b)
        cacc = lax.fori_loop(0, BW, commit_vec,
                             jnp.zeros((16,), jnp.float32))

        ones16 = jnp.ones((16,), jnp.float32)

        def hist_body(j, _):
            ivec = idx_v[pl.ds(j * 16, 16)]
            plsc.addupdate_scatter(
                hist_v, [jnp.right_shift(ivec, 7), ivec & 127], ones16)
            return 0
        lax.fori_loop(0, BW // 16, hist_body, 0)

        @pl.when(wid == 0)
        def _seed():
            pltpu.sync_copy(hist_v, counts_sh)
        plsc.subcore_barrier()

        @pl.when(wid != 0)
        def _add():
            for jj in range(K // 128 // 16):
                rows = lax.iota(jnp.int32, 16) + jj * 16
                pltpu.sync_copy(hist_v.at[pl.ds(jj * 16, 16)],
                                counts_sh.at[rows], add=True)
        stage_v[...] = cacc
        pltpu.sync_copy(stage_v, cpart_sh.at[wid])
        plsc.subcore_barrier()

        pltpu.sync_copy(counts_sh.at[pl.ds(wid * (BW // 128), BW // 128)],
                        cnt_v)

        def perp_vec(j, acc):
            cv = cnt_v[j >> 3, pl.ds((j & 7) * 16, 16)]
            ci = cv.astype(jnp.int32)
            tv = plsc.load_gather(tab_v, [ci])
            return acc + cv * tv
        pacc = lax.fori_loop(0, BW // 16, perp_vec,
                             jnp.zeros((16,), jnp.float32))
        stage_v[...] = pacc
        pltpu.sync_copy(stage_v, ppart_sh.at[wid])
        plsc.subcore_barrier()

        @pl.when(wid == 0)
        def _final():
            pltpu.sync_copy(cpart_sh, bulk_v)

            def red(j, acc):
                return acc + bulk_v[j, pl.ds(0, 16)]
            ca = lax.fori_loop(0, NW, red, jnp.zeros((16,), jnp.float32))
            pltpu.sync_copy(ppart_sh, bulk_v)
            pa = lax.fori_loop(0, NW, red, jnp.zeros((16,), jnp.float32))
            commit = jnp.sum(ca) * (1.0 / (N * D))
            stage_v[...] = jnp.broadcast_to(commit, (16,))
            pltpu.sync_copy(stage_v, commit_hbm)
            parg = jnp.broadcast_to(-jnp.sum(pa) * (1.0 / N), (16,))
            stage_v[...] = jnp.exp(parg)
            pltpu.sync_copy(stage_v, perp_hbm)

    return sc_kernel


def kernel(inputs, embedding, code_count):
    input_shape = inputs.shape
    flat_x = inputs.reshape(N, D)
    cc_col = code_count.reshape(K, 1)
    idx, scal = _vq_tc(flat_x, embedding, cc_col)
    tab = jnp.log(jnp.arange(TABN, dtype=jnp.float32) * (1.0 / N) + 1e-10)
    zz = jnp.zeros((K // 128, 128), jnp.float32)
    quantized, commit_v, perp_v = _make_sc_kernel()(embedding, idx, flat_x,
                                                    tab, zz)
    quantized_st = quantized.reshape(input_shape)
    commitment_loss = commit_v[0]
    ortho_loss = scal[1]
    entropy_loss = scal[2]
    perplexity = perp_v[0]
    coverage = scal[4]
    return (commitment_loss, ortho_loss, entropy_loss, perplexity, coverage,
            quantized_st, idx.reshape(input_shape[0], -1))
